# baseline (speedup 1.0000x reference)
import functools

import jax
import jax.numpy as jnp
from jax import lax
from jax.experimental import pallas as pl
from jax.experimental.pallas import tpu as pltpu
from jax.experimental.pallas import tpu_sc as plsc

E = 8
TOPK = 2
T = 256
NW = 32
CH = 64
CT = 32


def _router_body(x_ref, rw_ref, a1_ref, a2_ref, g1_ref, g2_ref):
    logits = lax.dot_general(
        x_ref[...], rw_ref[...], (((1,), (1,)), ((), ())),
        preferred_element_type=jnp.float32)
    bt = logits.shape[0]
    iot = lax.broadcasted_iota(jnp.int32, (bt, E), 1)
    m1 = jnp.max(logits, axis=1)
    a1 = jnp.min(jnp.where(logits == m1[:, None], iot, E), axis=1)
    masked = jnp.where(iot == a1[:, None], -jnp.inf, logits)
    m2 = jnp.max(masked, axis=1)
    a2 = jnp.min(jnp.where(masked == m2[:, None], iot, E), axis=1)
    d = jnp.exp(m2 - m1)
    a1_ref[...] = a1
    a2_ref[...] = a2
    g1_ref[...] = 1.0 / (1.0 + d)
    g2_ref[...] = d / (1.0 + d)


def _router(xf, rw):
    n, h = xf.shape
    bt = 512
    return pl.pallas_call(
        _router_body,
        grid=(n // bt,),
        in_specs=[pl.BlockSpec((bt, h), lambda t: (t, 0)),
                  pl.BlockSpec((E, h), lambda t: (0, 0))],
        out_specs=[pl.BlockSpec((bt,), lambda t: (t,)) for _ in range(4)],
        out_shape=[jax.ShapeDtypeStruct((n,), jnp.int32),
                   jax.ShapeDtypeStruct((n,), jnp.int32),
                   jax.ShapeDtypeStruct((n,), jnp.float32),
                   jax.ShapeDtypeStruct((n,), jnp.float32)],
    )(xf, rw)


def _dispatch_plan(a1, a2, g1, g2, n, p):
    fe = jnp.stack([a1, a2], axis=1).reshape(-1)
    oh = (fe[:, None] == jnp.arange(E, dtype=jnp.int32)[None, :]).astype(jnp.int32)
    csum = jnp.cumsum(oh, axis=0)
    rank = jnp.take_along_axis(csum, fe[:, None], axis=1)[:, 0] - 1
    counts = csum[-1]
    padded = ((counts + T - 1) // T) * T
    bounds = jnp.cumsum(padded)
    astart = bounds - padded
    slot = astart[fe] + rank
    rows_src = jnp.zeros((p,), jnp.int32).at[slot].set(
        jnp.arange(2 * n, dtype=jnp.int32) // TOPK)
    tile_start = jnp.arange(p // T, dtype=jnp.int32) * T
    tile_expert = jnp.minimum(
        jnp.searchsorted(bounds, tile_start, side="right"), E - 1
    ).astype(jnp.int32)
    gates = jnp.stack([g1, g2], axis=1).reshape(-1)
    gates_p = jnp.zeros((p,), jnp.float32).at[slot].set(gates)
    s01 = slot.reshape(n, TOPK)
    return rows_src, tile_expert, gates_p, s01[:, 0], s01[:, 1]


def _sc_dispatch(xf, rows_src, p):
    n, h = xf.shape
    per_w = p // NW
    nch = per_w // CH
    mesh = plsc.VectorSubcoreMesh(core_axis_name="c", subcore_axis_name="s")

    @functools.partial(
        pl.kernel, mesh=mesh,
        out_type=jax.ShapeDtypeStruct((p, h), jnp.float32),
        scratch_types=[pltpu.VMEM((CH,), jnp.int32),
                       pltpu.VMEM((CH, h), jnp.float32),
                       pltpu.SemaphoreType.DMA])
    def k(x_hbm, idx_hbm, out_hbm, idx_v, rows_v, sem):
        wid = lax.axis_index("s") * 2 + lax.axis_index("c")
        base = wid * per_w

        def body(i, carry):
            b = base + i * CH
            pltpu.sync_copy(idx_hbm.at[pl.ds(b, CH)], idx_v)
            pltpu.async_copy(x_hbm.at[idx_v], rows_v, sem).wait()
            pltpu.sync_copy(rows_v, out_hbm.at[pl.ds(b, CH)])
            return carry

        lax.fori_loop(0, nch, body, 0)

    return k(xf, rows_src)


def _ffn_body(te_ref, xs_ref, w1_ref, w2_ref, wo_ref, g_ref, y_ref, acc_ref):
    j = pl.program_id(1)
    nj = pl.num_programs(1)
    x16 = xs_ref[...].astype(jnp.bfloat16)
    a = lax.dot_general(x16, w1_ref[0].astype(jnp.bfloat16),
                        (((1,), (1,)), ((), ())),
                        preferred_element_type=jnp.float32)
    b = lax.dot_general(x16, w2_ref[0].astype(jnp.bfloat16),
                        (((1,), (1,)), ((), ())),
                        preferred_element_type=jnp.float32)
    g = (a * jax.nn.sigmoid(a) * b).astype(jnp.bfloat16)
    contrib = lax.dot_general(g, wo_ref[0].astype(jnp.bfloat16),
                              (((1,), (1,)), ((), ())),
                              preferred_element_type=jnp.float32)

    @pl.when(j == 0)
    def _():
        acc_ref[...] = contrib

    @pl.when(j > 0)
    def _():
        acc_ref[...] = acc_ref[...] + contrib

    @pl.when(j == nj - 1)
    def _():
        y_ref[...] = acc_ref[...] * g_ref[0, 0][:, None]


def _ffn(xs, w_in, w_out, gates_p, tile_expert, p):
    h = xs.shape[1]
    nt = p // T
    nj = 4
    grid_spec = pltpu.PrefetchScalarGridSpec(
        num_scalar_prefetch=1,
        grid=(nt, nj),
        in_specs=[
            pl.BlockSpec((T, h), lambda t, j, te: (t, 0)),
            pl.BlockSpec((1, 1024, h), lambda t, j, te: (te[t], j, 0)),
            pl.BlockSpec((1, 1024, h), lambda t, j, te: (te[t], j + nj, 0)),
            pl.BlockSpec((1, h, 1024), lambda t, j, te: (te[t], 0, j)),
            pl.BlockSpec((1, 1, T), lambda t, j, te: (t, 0, 0)),
        ],
        out_specs=pl.BlockSpec((T, h), lambda t, j, te: (t, 0)),
        scratch_shapes=[pltpu.VMEM((T, h), jnp.float32)],
    )
    return pl.pallas_call(
        _ffn_body, grid_spec=grid_spec,
        out_shape=jax.ShapeDtypeStruct((p, h), jnp.float32),
    )(tile_expert, xs, w_in, w_in, w_out, gates_p.reshape(nt, 1, T))


def _sc_combine(y, s0, s1, n, h):
    per_w = n // NW
    nch = per_w // CT
    mesh = plsc.VectorSubcoreMesh(core_axis_name="c", subcore_axis_name="s")

    @functools.partial(
        pl.kernel, mesh=mesh,
        out_type=jax.ShapeDtypeStruct((n, h), jnp.float32),
        scratch_types=[pltpu.VMEM((CT,), jnp.int32),
                       pltpu.VMEM((CT,), jnp.int32),
                       pltpu.VMEM((CT, h), jnp.float32),
                       pltpu.VMEM((CT, h), jnp.float32),
                       pltpu.SemaphoreType.DMA,
                       pltpu.SemaphoreType.DMA])
    def k(y_hbm, s0_hbm, s1_hbm, out_hbm, i0_v, i1_v, b0_v, b1_v, sem0, sem1):
        wid = lax.axis_index("s") * 2 + lax.axis_index("c")
        base = wid * per_w

        def body(i, carry):
            b = base + i * CT
            pltpu.sync_copy(s0_hbm.at[pl.ds(b, CT)], i0_v)
            pltpu.sync_copy(s1_hbm.at[pl.ds(b, CT)], i1_v)
            cp0 = pltpu.async_copy(y_hbm.at[i0_v], b0_v, sem0)
            cp1 = pltpu.async_copy(y_hbm.at[i1_v], b1_v, sem1)
            cp0.wait()
            cp1.wait()

            def add_row(t, c2):
                def add_lane(jj, c3):
                    sl = pl.ds(jj * 16, 16)
                    b0_v[t, sl] = b0_v[t, sl] + b1_v[t, sl]
                    return c3
                return lax.fori_loop(0, h // 16, add_lane, c2, unroll=8)

            lax.fori_loop(0, CT, add_row, 0)
            pltpu.sync_copy(b0_v, out_hbm.at[pl.ds(b, CT)])
            return carry

        lax.fori_loop(0, nch, body, 0)

    return k(y, s0, s1)


def kernel(x, router_weight, w_in, w_out):
    bsz, seq, h = x.shape
    n = bsz * seq
    p = TOPK * n + E * T
    xf = x.reshape(n, h)
    a1, a2, g1, g2 = _router(xf, router_weight)
    rows_src, tile_expert, gates_p, s0, s1 = _dispatch_plan(a1, a2, g1, g2, n, p)
    xs = _sc_dispatch(xf, rows_src, p)
    y = _ffn(xs, w_in, w_out, gates_p, tile_expert, p)
    out = _sc_combine(y, s0, s1, n, h)
    return out.reshape(bsz, seq, h)


# speedup vs baseline: 1.0870x; 1.0870x over previous
import functools

import jax
import jax.numpy as jnp
from jax import lax
from jax.experimental import pallas as pl
from jax.experimental.pallas import tpu as pltpu
from jax.experimental.pallas import tpu_sc as plsc

E = 8
TOPK = 2
T = 512
NW = 32
CH = 64
CT = 32


def _router_body(x_ref, rw_ref, a1_ref, a2_ref, g1_ref, g2_ref):
    logits = lax.dot_general(
        x_ref[...], rw_ref[...], (((1,), (1,)), ((), ())),
        preferred_element_type=jnp.float32)
    bt = logits.shape[0]
    iot = lax.broadcasted_iota(jnp.int32, (bt, E), 1)
    m1 = jnp.max(logits, axis=1)
    a1 = jnp.min(jnp.where(logits == m1[:, None], iot, E), axis=1)
    masked = jnp.where(iot == a1[:, None], -jnp.inf, logits)
    m2 = jnp.max(masked, axis=1)
    a2 = jnp.min(jnp.where(masked == m2[:, None], iot, E), axis=1)
    d = jnp.exp(m2 - m1)
    a1_ref[...] = a1
    a2_ref[...] = a2
    g1_ref[...] = 1.0 / (1.0 + d)
    g2_ref[...] = d / (1.0 + d)


def _router(xf, rw):
    n, h = xf.shape
    bt = 512
    return pl.pallas_call(
        _router_body,
        grid=(n // bt,),
        in_specs=[pl.BlockSpec((bt, h), lambda t: (t, 0)),
                  pl.BlockSpec((E, h), lambda t: (0, 0))],
        out_specs=[pl.BlockSpec((bt,), lambda t: (t,)) for _ in range(4)],
        out_shape=[jax.ShapeDtypeStruct((n,), jnp.int32),
                   jax.ShapeDtypeStruct((n,), jnp.int32),
                   jax.ShapeDtypeStruct((n,), jnp.float32),
                   jax.ShapeDtypeStruct((n,), jnp.float32)],
    )(xf, rw)


def _dispatch_plan(a1, a2, g1, g2, n, p):
    fe = jnp.stack([a1, a2], axis=1).reshape(-1)
    oh = (fe[:, None] == jnp.arange(E, dtype=jnp.int32)[None, :]).astype(jnp.int32)
    csum = jnp.cumsum(oh, axis=0)
    rank = jnp.take_along_axis(csum, fe[:, None], axis=1)[:, 0] - 1
    counts = csum[-1]
    padded = ((counts + T - 1) // T) * T
    bounds = jnp.cumsum(padded)
    astart = bounds - padded
    slot = astart[fe] + rank
    rows_src = jnp.zeros((p,), jnp.int32).at[slot].set(
        jnp.arange(2 * n, dtype=jnp.int32) // TOPK)
    tile_start = jnp.arange(p // T, dtype=jnp.int32) * T
    tile_expert = jnp.minimum(
        jnp.sum((tile_start[:, None] >= bounds[None, :]).astype(jnp.int32), axis=1),
        E - 1).astype(jnp.int32)
    gates = jnp.stack([g1, g2], axis=1).reshape(-1)
    gates_p = jnp.zeros((p,), jnp.float32).at[slot].set(gates)
    s01 = slot.reshape(n, TOPK)
    return rows_src, tile_expert, gates_p, s01[:, 0], s01[:, 1]


def _sc_dispatch(xf, rows_src, p):
    n, h = xf.shape
    per_w = p // NW
    nch = per_w // CH
    mesh = plsc.VectorSubcoreMesh(core_axis_name="c", subcore_axis_name="s")

    @functools.partial(
        pl.kernel, mesh=mesh,
        out_type=jax.ShapeDtypeStruct((p, h), jnp.float32),
        scratch_types=[pltpu.VMEM((CH,), jnp.int32),
                       pltpu.VMEM((CH, h), jnp.float32),
                       pltpu.SemaphoreType.DMA])
    def k(x_hbm, idx_hbm, out_hbm, idx_v, rows_v, sem):
        wid = lax.axis_index("s") * 2 + lax.axis_index("c")
        base = wid * per_w

        def body(i, carry):
            b = base + i * CH
            pltpu.sync_copy(idx_hbm.at[pl.ds(b, CH)], idx_v)
            pltpu.async_copy(x_hbm.at[idx_v], rows_v, sem).wait()
            pltpu.sync_copy(rows_v, out_hbm.at[pl.ds(b, CH)])
            return carry

        lax.fori_loop(0, nch, body, 0)

    return k(xf, rows_src)


def _ffn_body(te_ref, xs_ref, w1_ref, w2_ref, wo_ref, g_ref, y_ref, acc_ref):
    j = pl.program_id(1)
    nj = pl.num_programs(1)
    x16 = xs_ref[...].astype(jnp.bfloat16)
    a = lax.dot_general(x16, w1_ref[0].astype(jnp.bfloat16),
                        (((1,), (1,)), ((), ())),
                        preferred_element_type=jnp.float32)
    b = lax.dot_general(x16, w2_ref[0].astype(jnp.bfloat16),
                        (((1,), (1,)), ((), ())),
                        preferred_element_type=jnp.float32)
    g = (a * jax.nn.sigmoid(a) * b).astype(jnp.bfloat16)
    contrib = lax.dot_general(g, wo_ref[0].astype(jnp.bfloat16),
                              (((1,), (1,)), ((), ())),
                              preferred_element_type=jnp.float32)

    @pl.when(j == 0)
    def _():
        acc_ref[...] = contrib

    @pl.when(j > 0)
    def _():
        acc_ref[...] = acc_ref[...] + contrib

    @pl.when(j == nj - 1)
    def _():
        y_ref[...] = acc_ref[...] * g_ref[0, 0][:, None]


def _ffn(xs, w_in, w_out, gates_p, tile_expert, p):
    h = xs.shape[1]
    nt = p // T
    nj = 4
    grid_spec = pltpu.PrefetchScalarGridSpec(
        num_scalar_prefetch=1,
        grid=(nt, nj),
        in_specs=[
            pl.BlockSpec((T, h), lambda t, j, te: (t, 0)),
            pl.BlockSpec((1, 1024, h), lambda t, j, te: (te[t], j, 0)),
            pl.BlockSpec((1, 1024, h), lambda t, j, te: (te[t], j + nj, 0)),
            pl.BlockSpec((1, h, 1024), lambda t, j, te: (te[t], 0, j)),
            pl.BlockSpec((1, 1, T), lambda t, j, te: (t, 0, 0)),
        ],
        out_specs=pl.BlockSpec((T, h), lambda t, j, te: (t, 0)),
        scratch_shapes=[pltpu.VMEM((T, h), jnp.float32)],
    )
    return pl.pallas_call(
        _ffn_body, grid_spec=grid_spec,
        out_shape=jax.ShapeDtypeStruct((p, h), jnp.float32),
    )(tile_expert, xs, w_in, w_in, w_out, gates_p.reshape(nt, 1, T))


def _sc_combine(y, s0, s1, n, h):
    per_w = n // NW
    nch = per_w // CT
    mesh = plsc.VectorSubcoreMesh(core_axis_name="c", subcore_axis_name="s")

    @functools.partial(
        pl.kernel, mesh=mesh,
        out_type=jax.ShapeDtypeStruct((n, h), jnp.float32),
        scratch_types=[pltpu.VMEM((CT,), jnp.int32),
                       pltpu.VMEM((CT,), jnp.int32),
                       pltpu.VMEM((CT, h), jnp.float32),
                       pltpu.VMEM((CT, h), jnp.float32),
                       pltpu.SemaphoreType.DMA,
                       pltpu.SemaphoreType.DMA])
    def k(y_hbm, s0_hbm, s1_hbm, out_hbm, i0_v, i1_v, b0_v, b1_v, sem0, sem1):
        wid = lax.axis_index("s") * 2 + lax.axis_index("c")
        base = wid * per_w

        def body(i, carry):
            b = base + i * CT
            pltpu.sync_copy(s0_hbm.at[pl.ds(b, CT)], i0_v)
            pltpu.sync_copy(s1_hbm.at[pl.ds(b, CT)], i1_v)
            cp0 = pltpu.async_copy(y_hbm.at[i0_v], b0_v, sem0)
            cp1 = pltpu.async_copy(y_hbm.at[i1_v], b1_v, sem1)
            cp0.wait()
            cp1.wait()

            def add_row(t, c2):
                def add_lane(jj, c3):
                    sl = pl.ds(jj * 16, 16)
                    b0_v[t, sl] = b0_v[t, sl] + b1_v[t, sl]
                    return c3
                return lax.fori_loop(0, h // 16, add_lane, c2, unroll=8)

            lax.fori_loop(0, CT, add_row, 0)
            pltpu.sync_copy(b0_v, out_hbm.at[pl.ds(b, CT)])
            return carry

        lax.fori_loop(0, nch, body, 0)

    return k(y, s0, s1)


def kernel(x, router_weight, w_in, w_out):
    bsz, seq, h = x.shape
    n = bsz * seq
    p = TOPK * n + E * T
    xf = x.reshape(n, h)
    a1, a2, g1, g2 = _router(xf, router_weight)
    rows_src, tile_expert, gates_p, s0, s1 = _dispatch_plan(a1, a2, g1, g2, n, p)
    xs = _sc_dispatch(xf, rows_src, p)
    y = _ffn(xs, w_in, w_out, gates_p, tile_expert, p)
    out = _sc_combine(y, s0, s1, n, h)
    return out.reshape(bsz, seq, h)


# speedup vs baseline: 1.5114x; 1.3905x over previous
import functools

import jax
import jax.numpy as jnp
from jax import lax
from jax.experimental import pallas as pl
from jax.experimental.pallas import tpu as pltpu
from jax.experimental.pallas import tpu_sc as plsc

E = 8
TOPK = 2
T = 512
NW = 32
CD = 32
CT = 16


def _router_body(x_ref, rw_ref, a1_ref, a2_ref, g1_ref, g2_ref):
    logits = lax.dot_general(
        x_ref[...], rw_ref[...], (((1,), (1,)), ((), ())),
        preferred_element_type=jnp.float32)
    bt = logits.shape[0]
    iot = lax.broadcasted_iota(jnp.int32, (bt, E), 1)
    m1 = jnp.max(logits, axis=1)
    a1 = jnp.min(jnp.where(logits == m1[:, None], iot, E), axis=1)
    masked = jnp.where(iot == a1[:, None], -jnp.inf, logits)
    m2 = jnp.max(masked, axis=1)
    a2 = jnp.min(jnp.where(masked == m2[:, None], iot, E), axis=1)
    d = jnp.exp(m2 - m1)
    a1_ref[...] = a1
    a2_ref[...] = a2
    g1_ref[...] = 1.0 / (1.0 + d)
    g2_ref[...] = d / (1.0 + d)


def _router(xf, rw):
    n, h = xf.shape
    bt = 512
    return pl.pallas_call(
        _router_body,
        grid=(n // bt,),
        in_specs=[pl.BlockSpec((bt, h), lambda t: (t, 0)),
                  pl.BlockSpec((E, h), lambda t: (0, 0))],
        out_specs=[pl.BlockSpec((bt,), lambda t: (t,)) for _ in range(4)],
        out_shape=[jax.ShapeDtypeStruct((n,), jnp.int32),
                   jax.ShapeDtypeStruct((n,), jnp.int32),
                   jax.ShapeDtypeStruct((n,), jnp.float32),
                   jax.ShapeDtypeStruct((n,), jnp.float32)],
    )(xf, rw)


def _dispatch_plan(a1, a2, g1, g2, n, p):
    fe = jnp.stack([a1, a2], axis=1).reshape(-1)
    oh = (fe[:, None] == jnp.arange(E, dtype=jnp.int32)[None, :]).astype(jnp.int32)
    csum = jnp.cumsum(oh, axis=0)
    rank = jnp.take_along_axis(csum, fe[:, None], axis=1)[:, 0] - 1
    counts = csum[-1]
    padded = ((counts + T - 1) // T) * T
    bounds = jnp.cumsum(padded)
    astart = bounds - padded
    slot = astart[fe] + rank
    tile_start = jnp.arange(p // T, dtype=jnp.int32) * T
    tile_expert = jnp.minimum(
        jnp.sum((tile_start[:, None] >= bounds[None, :]).astype(jnp.int32), axis=1),
        E - 1).astype(jnp.int32)
    s01 = slot.reshape(n, TOPK)
    return tile_expert, s01[:, 0], s01[:, 1]


def _sc_dispatch(xf, s0, s1, p):
    n, h = xf.shape
    tpw = n // NW
    nch = tpw // CD
    mesh = plsc.VectorSubcoreMesh(core_axis_name="c", subcore_axis_name="s")

    @functools.partial(
        pl.kernel, mesh=mesh,
        out_type=jax.ShapeDtypeStruct((p, h), jnp.float32),
        scratch_types=[[pltpu.VMEM((CD, h), jnp.float32)] * 2,
                       [pltpu.VMEM((CD,), jnp.int32)] * 2,
                       [pltpu.VMEM((CD,), jnp.int32)] * 2,
                       [pltpu.SemaphoreType.DMA] * 2,
                       [pltpu.SemaphoreType.DMA] * 2])
    def k(x_hbm, s0_hbm, s1_hbm, out_hbm, bufs, i0s, i1s, sem0s, sem1s):
        wid = lax.axis_index("s") * 2 + lax.axis_index("c")
        base = wid * tpw
        pend = [None, None]
        for i in range(nch):
            pp = i % 2
            if i >= 2:
                pend[pp][0].wait()
                pend[pp][1].wait()
            b = base + i * CD
            pltpu.sync_copy(x_hbm.at[pl.ds(b, CD)], bufs[pp])
            pltpu.sync_copy(s0_hbm.at[pl.ds(b, CD)], i0s[pp])
            pltpu.sync_copy(s1_hbm.at[pl.ds(b, CD)], i1s[pp])
            pend[pp] = (
                pltpu.async_copy(bufs[pp], out_hbm.at[i0s[pp]], sem0s[pp]),
                pltpu.async_copy(bufs[pp], out_hbm.at[i1s[pp]], sem1s[pp]))
        for pp in range(min(2, nch)):
            pend[pp][0].wait()
            pend[pp][1].wait()

    return k(xf, s0, s1)


def _ffn_body(te_ref, xs_ref, w1_ref, w2_ref, wo_ref, y_ref, acc_ref):
    j = pl.program_id(1)
    nj = pl.num_programs(1)
    x16 = xs_ref[...].astype(jnp.bfloat16)
    a = lax.dot_general(x16, w1_ref[0].astype(jnp.bfloat16),
                        (((1,), (1,)), ((), ())),
                        preferred_element_type=jnp.float32)
    b = lax.dot_general(x16, w2_ref[0].astype(jnp.bfloat16),
                        (((1,), (1,)), ((), ())),
                        preferred_element_type=jnp.float32)
    g = (a * jax.nn.sigmoid(a) * b).astype(jnp.bfloat16)
    contrib = lax.dot_general(g, wo_ref[0].astype(jnp.bfloat16),
                              (((1,), (1,)), ((), ())),
                              preferred_element_type=jnp.float32)

    @pl.when(j == 0)
    def _():
        acc_ref[...] = contrib

    @pl.when(j > 0)
    def _():
        acc_ref[...] = acc_ref[...] + contrib

    @pl.when(j == nj - 1)
    def _():
        y_ref[...] = acc_ref[...]


def _ffn(xs, w_in, w_out, tile_expert, p):
    h = xs.shape[1]
    nt = p // T
    nj = 4
    grid_spec = pltpu.PrefetchScalarGridSpec(
        num_scalar_prefetch=1,
        grid=(nt, nj),
        in_specs=[
            pl.BlockSpec((T, h), lambda t, j, te: (t, 0)),
            pl.BlockSpec((1, 1024, h), lambda t, j, te: (te[t], j, 0)),
            pl.BlockSpec((1, 1024, h), lambda t, j, te: (te[t], j + nj, 0)),
            pl.BlockSpec((1, h, 1024), lambda t, j, te: (te[t], 0, j)),
        ],
        out_specs=pl.BlockSpec((T, h), lambda t, j, te: (t, 0)),
        scratch_shapes=[pltpu.VMEM((T, h), jnp.float32)],
    )
    return pl.pallas_call(
        _ffn_body, grid_spec=grid_spec,
        out_shape=jax.ShapeDtypeStruct((p, h), jnp.float32),
    )(tile_expert, xs, w_in, w_in, w_out)


def _sc_combine(y, s0, s1, g0, g1, n, h):
    per_w = n // NW
    nch = per_w // CT
    mesh = plsc.VectorSubcoreMesh(core_axis_name="c", subcore_axis_name="s")

    @functools.partial(
        pl.kernel, mesh=mesh,
        out_type=jax.ShapeDtypeStruct((n, h), jnp.float32),
        scratch_types=[[pltpu.VMEM((CT,), jnp.int32)] * 2,
                       [pltpu.VMEM((CT,), jnp.int32)] * 2,
                       [pltpu.VMEM((CT,), jnp.float32)] * 2,
                       [pltpu.VMEM((CT,), jnp.float32)] * 2,
                       [pltpu.VMEM((CT, h), jnp.float32)] * 2,
                       [pltpu.VMEM((CT, h), jnp.float32)] * 2,
                       [pltpu.SemaphoreType.DMA] * 2,
                       [pltpu.SemaphoreType.DMA] * 2,
                       [pltpu.SemaphoreType.DMA] * 2])
    def k(y_hbm, s0_hbm, s1_hbm, g0_hbm, g1_hbm, out_hbm,
          i0s, i1s, g0s, g1s, b0s, b1s, rs0, rs1, ws):
        wid = lax.axis_index("s") * 2 + lax.axis_index("c")
        base = wid * per_w
        rd = [None, None]
        wr = [None, None]

        def load(i):
            pp = i % 2
            b = base + i * CT
            pltpu.sync_copy(s0_hbm.at[pl.ds(b, CT)], i0s[pp])
            pltpu.sync_copy(s1_hbm.at[pl.ds(b, CT)], i1s[pp])
            pltpu.sync_copy(g0_hbm.at[pl.ds(b, CT)], g0s[pp])
            pltpu.sync_copy(g1_hbm.at[pl.ds(b, CT)], g1s[pp])
            rd[pp] = (pltpu.async_copy(y_hbm.at[i0s[pp]], b0s[pp], rs0[pp]),
                      pltpu.async_copy(y_hbm.at[i1s[pp]], b1s[pp], rs1[pp]))

        load(0)
        for i in range(nch):
            pp = i % 2
            if i + 1 < nch:
                if i >= 1:
                    wr[(i + 1) % 2].wait()
                load(i + 1)
            rd[pp][0].wait()
            rd[pp][1].wait()

            gv0 = g0s[pp][...]
            gv1 = g1s[pp][...]
            dn = lax.GatherDimensionNumbers(
                offset_dims=(), collapsed_slice_dims=(0,), start_index_map=(0,))

            def fma_row(t, c2):
                tv = jnp.full((CT, 1), t, jnp.int32)
                ga = lax.gather(gv0, tv, dn, (1,),
                                mode=lax.GatherScatterMode.PROMISE_IN_BOUNDS)
                gb = lax.gather(gv1, tv, dn, (1,),
                                mode=lax.GatherScatterMode.PROMISE_IN_BOUNDS)

                def fma_lane(jj, c3):
                    sl = pl.ds(jj * 16, 16)
                    b0s[pp][t, sl] = b0s[pp][t, sl] * ga + b1s[pp][t, sl] * gb
                    return c3
                return lax.fori_loop(0, h // 16, fma_lane, c2, unroll=8)

            lax.fori_loop(0, CT, fma_row, 0)
            wr[pp] = pltpu.async_copy(
                b0s[pp], out_hbm.at[pl.ds(base + i * CT, CT)], ws[pp])
        for pp in range(min(2, nch)):
            wr[(nch - 1 - pp) % 2].wait()

    return k(y, s0, s1, g0, g1)


def kernel(x, router_weight, w_in, w_out):
    bsz, seq, h = x.shape
    n = bsz * seq
    p = TOPK * n + E * T
    xf = x.reshape(n, h)
    a1, a2, g1, g2 = _router(xf, router_weight)
    tile_expert, s0, s1 = _dispatch_plan(a1, a2, g1, g2, n, p)
    xs = _sc_dispatch(xf, s0, s1, p)
    y = _ffn(xs, w_in, w_out, tile_expert, p)
    out = _sc_combine(y, s0, s1, g1, g2, n, h)
    return out.reshape(bsz, seq, h)


# speedup vs baseline: 1.5745x; 1.0418x over previous
import functools

import jax
import jax.numpy as jnp
from jax import lax
from jax.experimental import pallas as pl
from jax.experimental.pallas import tpu as pltpu
from jax.experimental.pallas import tpu_sc as plsc

E = 8
TOPK = 2
T = 512
NW = 32
CD = 32
CT = 16


def _router_body(x_ref, rw_ref, a1_ref, a2_ref, g1_ref, g2_ref):
    logits = lax.dot_general(
        x_ref[...], rw_ref[...], (((1,), (1,)), ((), ())),
        preferred_element_type=jnp.float32)
    bt = logits.shape[0]
    iot = lax.broadcasted_iota(jnp.int32, (bt, E), 1)
    m1 = jnp.max(logits, axis=1)
    a1 = jnp.min(jnp.where(logits == m1[:, None], iot, E), axis=1)
    masked = jnp.where(iot == a1[:, None], -jnp.inf, logits)
    m2 = jnp.max(masked, axis=1)
    a2 = jnp.min(jnp.where(masked == m2[:, None], iot, E), axis=1)
    d = jnp.exp(m2 - m1)
    a1_ref[...] = a1
    a2_ref[...] = a2
    g1_ref[...] = 1.0 / (1.0 + d)
    g2_ref[...] = d / (1.0 + d)


def _router(xf, rw):
    n, h = xf.shape
    bt = 512
    return pl.pallas_call(
        _router_body,
        grid=(n // bt,),
        in_specs=[pl.BlockSpec((bt, h), lambda t: (t, 0)),
                  pl.BlockSpec((E, h), lambda t: (0, 0))],
        out_specs=[pl.BlockSpec((bt,), lambda t: (t,)) for _ in range(4)],
        out_shape=[jax.ShapeDtypeStruct((n,), jnp.int32),
                   jax.ShapeDtypeStruct((n,), jnp.int32),
                   jax.ShapeDtypeStruct((n,), jnp.float32),
                   jax.ShapeDtypeStruct((n,), jnp.float32)],
    )(xf, rw)


def _dispatch_plan(a1, a2, g1, g2, n, p):
    fe = jnp.stack([a1, a2], axis=1).reshape(-1)
    oh = (fe[:, None] == jnp.arange(E, dtype=jnp.int32)[None, :]).astype(jnp.int32)
    csum = jnp.cumsum(oh, axis=0)
    rank = jnp.sum(csum * oh, axis=1) - 1
    counts = csum[-1]
    padded = ((counts + T - 1) // T) * T
    bounds = jnp.cumsum(padded)
    astart = bounds - padded
    slot = jnp.sum(astart[None, :] * oh, axis=1) + rank
    tile_start = jnp.arange(p // T, dtype=jnp.int32) * T
    tile_expert = jnp.minimum(
        jnp.sum((tile_start[:, None] >= bounds[None, :]).astype(jnp.int32), axis=1),
        E - 1).astype(jnp.int32)
    s01 = slot.reshape(n, TOPK)
    return tile_expert, s01[:, 0], s01[:, 1]


def _sc_dispatch(xf, s0, s1, p):
    n, h = xf.shape
    tpw = n // NW
    nch = tpw // CD
    mesh = plsc.VectorSubcoreMesh(core_axis_name="c", subcore_axis_name="s")

    @functools.partial(
        pl.kernel, mesh=mesh,
        out_type=jax.ShapeDtypeStruct((p, h), jnp.float32),
        scratch_types=[[pltpu.VMEM((CD, h), jnp.float32)] * 2,
                       [pltpu.VMEM((CD,), jnp.int32)] * 2,
                       [pltpu.VMEM((CD,), jnp.int32)] * 2,
                       [pltpu.SemaphoreType.DMA] * 2,
                       [pltpu.SemaphoreType.DMA] * 2])
    def k(x_hbm, s0_hbm, s1_hbm, out_hbm, bufs, i0s, i1s, sem0s, sem1s):
        wid = lax.axis_index("s") * 2 + lax.axis_index("c")
        base = wid * tpw
        pend = [None, None]
        for i in range(nch):
            pp = i % 2
            if i >= 2:
                pend[pp][0].wait()
                pend[pp][1].wait()
            b = base + i * CD
            pltpu.sync_copy(x_hbm.at[pl.ds(b, CD)], bufs[pp])
            pltpu.sync_copy(s0_hbm.at[pl.ds(b, CD)], i0s[pp])
            pltpu.sync_copy(s1_hbm.at[pl.ds(b, CD)], i1s[pp])
            pend[pp] = (
                pltpu.async_copy(bufs[pp], out_hbm.at[i0s[pp]], sem0s[pp]),
                pltpu.async_copy(bufs[pp], out_hbm.at[i1s[pp]], sem1s[pp]))
        for pp in range(min(2, nch)):
            pend[pp][0].wait()
            pend[pp][1].wait()

    return k(xf, s0, s1)


def _ffn_body(te_ref, xs_ref, w1_ref, w2_ref, wo_ref, y_ref, acc_ref):
    j = pl.program_id(1)
    nj = pl.num_programs(1)
    x16 = xs_ref[...].astype(jnp.bfloat16)
    a = lax.dot_general(x16, w1_ref[0].astype(jnp.bfloat16),
                        (((1,), (1,)), ((), ())),
                        preferred_element_type=jnp.float32)
    b = lax.dot_general(x16, w2_ref[0].astype(jnp.bfloat16),
                        (((1,), (1,)), ((), ())),
                        preferred_element_type=jnp.float32)
    g = (a * jax.nn.sigmoid(a) * b).astype(jnp.bfloat16)
    contrib = lax.dot_general(g, wo_ref[0].astype(jnp.bfloat16),
                              (((1,), (1,)), ((), ())),
                              preferred_element_type=jnp.float32)

    @pl.when(j == 0)
    def _():
        acc_ref[...] = contrib

    @pl.when(j > 0)
    def _():
        acc_ref[...] = acc_ref[...] + contrib

    @pl.when(j == nj - 1)
    def _():
        y_ref[...] = acc_ref[...]


def _ffn(xs, w_in, w_out, tile_expert, p):
    h = xs.shape[1]
    nt = p // T
    nj = 4
    grid_spec = pltpu.PrefetchScalarGridSpec(
        num_scalar_prefetch=1,
        grid=(nt, nj),
        in_specs=[
            pl.BlockSpec((T, h), lambda t, j, te: (t, 0)),
            pl.BlockSpec((1, 1024, h), lambda t, j, te: (te[t], j, 0)),
            pl.BlockSpec((1, 1024, h), lambda t, j, te: (te[t], j + nj, 0)),
            pl.BlockSpec((1, h, 1024), lambda t, j, te: (te[t], 0, j)),
        ],
        out_specs=pl.BlockSpec((T, h), lambda t, j, te: (t, 0)),
        scratch_shapes=[pltpu.VMEM((T, h), jnp.float32)],
    )
    return pl.pallas_call(
        _ffn_body, grid_spec=grid_spec,
        out_shape=jax.ShapeDtypeStruct((p, h), jnp.float32),
        compiler_params=pltpu.CompilerParams(
            dimension_semantics=("parallel", "arbitrary")),
    )(tile_expert, xs, w_in, w_in, w_out)


def _sc_combine(y, s0, s1, g0, g1, n, h):
    per_w = n // NW
    nch = per_w // CT
    mesh = plsc.VectorSubcoreMesh(core_axis_name="c", subcore_axis_name="s")

    @functools.partial(
        pl.kernel, mesh=mesh,
        out_type=jax.ShapeDtypeStruct((n, h), jnp.float32),
        scratch_types=[pltpu.VMEM((per_w,), jnp.int32),
                       pltpu.VMEM((per_w,), jnp.int32),
                       pltpu.VMEM((per_w,), jnp.float32),
                       pltpu.VMEM((per_w,), jnp.float32),
                       [pltpu.VMEM((CT, h), jnp.float32)] * 2,
                       [pltpu.VMEM((CT, h), jnp.float32)] * 2,
                       [pltpu.SemaphoreType.DMA] * 2,
                       [pltpu.SemaphoreType.DMA] * 2,
                       [pltpu.SemaphoreType.DMA] * 2])
    def k(y_hbm, s0_hbm, s1_hbm, g0_hbm, g1_hbm, out_hbm,
          i0a, i1a, g0a, g1a, b0s, b1s, rs0, rs1, ws):
        wid = lax.axis_index("s") * 2 + lax.axis_index("c")
        base = wid * per_w
        pltpu.sync_copy(s0_hbm.at[pl.ds(base, per_w)], i0a)
        pltpu.sync_copy(s1_hbm.at[pl.ds(base, per_w)], i1a)
        pltpu.sync_copy(g0_hbm.at[pl.ds(base, per_w)], g0a)
        pltpu.sync_copy(g1_hbm.at[pl.ds(base, per_w)], g1a)
        rd = [None, None]
        wr = [None, None]

        def load(i):
            pp = i % 2
            sl = pl.ds(i * CT, CT)
            rd[pp] = (pltpu.async_copy(y_hbm.at[i0a.at[sl]], b0s[pp], rs0[pp]),
                      pltpu.async_copy(y_hbm.at[i1a.at[sl]], b1s[pp], rs1[pp]))

        load(0)
        dn = lax.GatherDimensionNumbers(
            offset_dims=(), collapsed_slice_dims=(0,), start_index_map=(0,))
        for i in range(nch):
            pp = i % 2
            if i + 1 < nch:
                if i >= 1:
                    wr[(i + 1) % 2].wait()
                load(i + 1)
            rd[pp][0].wait()
            rd[pp][1].wait()

            gv0 = g0a[pl.ds(i * CT, CT)]
            gv1 = g1a[pl.ds(i * CT, CT)]

            def fma_row(t, c2):
                tv = jnp.full((CT, 1), t, jnp.int32)
                ga = lax.gather(gv0, tv, dn, (1,),
                                mode=lax.GatherScatterMode.PROMISE_IN_BOUNDS)
                gb = lax.gather(gv1, tv, dn, (1,),
                                mode=lax.GatherScatterMode.PROMISE_IN_BOUNDS)

                def fma_lane(jj, c3):
                    sl = pl.ds(jj * 16, 16)
                    b0s[pp][t, sl] = b0s[pp][t, sl] * ga + b1s[pp][t, sl] * gb
                    return c3
                return lax.fori_loop(0, h // 16, fma_lane, c2, unroll=8)

            lax.fori_loop(0, CT, fma_row, 0)
            wr[pp] = pltpu.async_copy(
                b0s[pp], out_hbm.at[pl.ds(base + i * CT, CT)], ws[pp])
        for pp in range(min(2, nch)):
            wr[(nch - 1 - pp) % 2].wait()

    return k(y, s0, s1, g0, g1)


def kernel(x, router_weight, w_in, w_out):
    bsz, seq, h = x.shape
    n = bsz * seq
    p = TOPK * n + E * T
    xf = x.reshape(n, h)
    a1, a2, g1, g2 = _router(xf, router_weight)
    tile_expert, s0, s1 = _dispatch_plan(a1, a2, g1, g2, n, p)
    xs = _sc_dispatch(xf, s0, s1, p)
    y = _ffn(xs, w_in, w_out, tile_expert, p)
    out = _sc_combine(y, s0, s1, g1, g2, n, h)
    return out.reshape(bsz, seq, h)
